# baseline (speedup 1.0000x reference)
import functools

import jax
import jax.numpy as jnp
from jax import lax
from jax.experimental import pallas as pl
from jax.experimental.pallas import tpu as pltpu
from jax.experimental.pallas import tpu_sc as plsc

_N = 10000
_D = 128
_H = 64
_Z = 32
_E = 320000
_EPS = 1e-3

_NW = 32
_CHUNK = 80
_CH = 125
_EPW = _CH * _CHUNK
_RPT = _N // 16


def _tc1_body(x_ref, w_ref, g_ref, v_ref, o_ref):
    scale = g_ref[...] * lax.rsqrt(v_ref[...] + _EPS)
    o_ref[...] = jnp.dot(x_ref[...], w_ref[...] * scale,
                         preferred_element_type=jnp.float32)


def _tc2_body(am_ref, ad_ref, w2_ref, g1_ref, b1_ref, m1_ref, v1_ref,
              g2_ref, v2_ref, o_ref):
    acc = am_ref[:, 0:_H] + am_ref[:, _H:2 * _H]
    deg = ad_ref[:, 0:1] + ad_ref[:, 8:9]
    inv = jnp.where(deg > 0, 1.0 / deg, 0.0)
    s1 = g1_ref[...] * lax.rsqrt(v1_ref[...] + _EPS)
    sh1 = b1_ref[...] - m1_ref[...] * s1
    h = jnp.maximum(acc * inv + sh1, 0.0)
    s2 = g2_ref[...] * lax.rsqrt(v2_ref[...] + _EPS)
    o_ref[...] = jnp.dot(h, w2_ref[...] * s2,
                         preferred_element_type=jnp.float32)


def _tc3_body(az_ref, ad_ref, g2_ref, b2_ref, m2_ref, v2_ref, o_ref):
    acc = az_ref[:, 0:_Z] + az_ref[:, _Z:2 * _Z]
    deg = ad_ref[:, 0:1] + ad_ref[:, 8:9]
    inv = jnp.where(deg > 0, 1.0 / deg, 0.0)
    s2 = g2_ref[...] * lax.rsqrt(v2_ref[...] + _EPS)
    sh2 = b2_ref[...] - m2_ref[...] * s2
    o_ref[...] = acc * inv + sh2


_mesh = plsc.VectorSubcoreMesh(core_axis_name="c", subcore_axis_name="s")


def _scatter_loop(src_s, cidx, bufa, bufb, sema, semb, issue_fn, wait_fn):
    pltpu.async_copy(src_s.at[cidx.at[pl.ds(0, _CHUNK)]], bufa, sema)
    pltpu.async_copy(src_s.at[cidx.at[pl.ds(_CHUNK, _CHUNK)]], bufb, semb)

    def body(i, carry):
        j0 = 2 * i
        pltpu.make_async_copy(
            src_s.at[cidx.at[pl.ds(j0 * _CHUNK, _CHUNK)]], bufa, sema).wait()
        issue_fn(bufa, j0)

        j1 = j0 + 1
        pltpu.make_async_copy(
            src_s.at[cidx.at[pl.ds(j1 * _CHUNK, _CHUNK)]], bufb, semb).wait()
        issue_fn(bufb, j1)

        wait_fn(bufa, j0)

        @pl.when(j0 + 2 < _CH)
        def _():
            pltpu.async_copy(
                src_s.at[cidx.at[pl.ds((j0 + 2) * _CHUNK, _CHUNK)]],
                bufa, sema)

        wait_fn(bufb, j1)

        @pl.when(j1 + 2 < _CH)
        def _():
            pltpu.async_copy(
                src_s.at[cidx.at[pl.ds((j1 + 2) * _CHUNK, _CHUNK)]],
                bufb, semb)

        return carry

    lax.fori_loop(0, _CH // 2, body, 0)
    if _CH % 2:
        j = _CH - 1
        pltpu.make_async_copy(
            src_s.at[cidx.at[pl.ds(j * _CHUNK, _CHUNK)]], bufa, sema).wait()
        issue_fn(bufa, j)
        wait_fn(bufa, j)


@functools.partial(
    pl.kernel,
    out_type=(
        jax.ShapeDtypeStruct((_N, 2 * _H), jnp.float32),
        jax.ShapeDtypeStruct((_N, 16), jnp.float32),
    ),
    mesh=_mesh,
    scratch_types=[
        pltpu.VMEM((_EPW,), jnp.int32),
        pltpu.VMEM((_EPW,), jnp.int32),
        pltpu.VMEM((_CHUNK, _H), jnp.float32),
        pltpu.VMEM((_CHUNK, _H), jnp.float32),
        pltpu.VMEM((_CHUNK, 8), jnp.float32),
        pltpu.VMEM_SHARED((_N, _H), jnp.float32),
        pltpu.VMEM_SHARED((_N, _H), jnp.float32),
        pltpu.VMEM_SHARED((_N, 8), jnp.float32),
        pltpu.SemaphoreType.DMA,
        pltpu.SemaphoreType.DMA,
        pltpu.SemaphoreType.DMA,
        pltpu.SemaphoreType.DMA,
    ],
    compiler_params=pltpu.CompilerParams(use_tc_tiling_on_sc=False),
)
def _sc_layer1(edges_hbm, xw_hbm, zrow_hbm, zdeg_hbm, ones_hbm,
               outm_hbm, outd_hbm,
               ridx, cidx, bufa, bufb, ones_v, xw_s, accm, accd,
               sema, semb, semsa, semsb):
    cid = lax.axis_index("c")
    sid = lax.axis_index("s")
    w = sid * 2 + cid
    e0 = w * _EPW
    r0 = sid * _RPT
    pltpu.sync_copy(zrow_hbm, accm.at[pl.ds(r0, _RPT)])
    pltpu.sync_copy(zdeg_hbm, accd.at[pl.ds(r0, _RPT)])
    pltpu.sync_copy(xw_hbm.at[pl.ds(r0, _RPT)], xw_s.at[pl.ds(r0, _RPT)])
    pltpu.sync_copy(edges_hbm.at[0, pl.ds(e0, _EPW)], ridx)
    pltpu.sync_copy(edges_hbm.at[1, pl.ds(e0, _EPW)], cidx)
    pltpu.sync_copy(ones_hbm, ones_v)
    plsc.subcore_barrier()

    def _sem(buf):
        return semsa if buf is bufa else semsb

    def issue(buf, j):
        dst = ridx.at[pl.ds(j * _CHUNK, _CHUNK)]
        pltpu.async_copy(buf, accm.at[dst], _sem(buf), add=True)
        pltpu.async_copy(ones_v, accd.at[dst], _sem(buf), add=True)

    def drain(buf, j):
        dst = ridx.at[pl.ds(j * _CHUNK, _CHUNK)]
        pltpu.make_async_copy(buf, accm.at[dst], _sem(buf)).wait()
        pltpu.make_async_copy(ones_v, accd.at[dst], _sem(buf)).wait()

    _scatter_loop(xw_s, cidx, bufa, bufb, sema, semb, issue, drain)
    plsc.subcore_barrier()
    pltpu.sync_copy(accm.at[pl.ds(r0, _RPT)],
                    outm_hbm.at[pl.ds(r0, _RPT), pl.ds(cid * _H, _H)])
    pltpu.sync_copy(accd.at[pl.ds(r0, _RPT)],
                    outd_hbm.at[pl.ds(r0, _RPT), pl.ds(cid * 8, 8)])


@functools.partial(
    pl.kernel,
    out_type=jax.ShapeDtypeStruct((_N, 2 * _Z), jnp.float32),
    mesh=_mesh,
    scratch_types=[
        pltpu.VMEM((_EPW,), jnp.int32),
        pltpu.VMEM((_EPW,), jnp.int32),
        pltpu.VMEM((_CHUNK, _Z), jnp.float32),
        pltpu.VMEM((_CHUNK, _Z), jnp.float32),
        pltpu.VMEM_SHARED((_N, _Z), jnp.float32),
        pltpu.VMEM_SHARED((_N, _Z), jnp.float32),
        pltpu.SemaphoreType.DMA,
        pltpu.SemaphoreType.DMA,
        pltpu.SemaphoreType.DMA,
        pltpu.SemaphoreType.DMA,
    ],
    compiler_params=pltpu.CompilerParams(use_tc_tiling_on_sc=False),
)
def _sc_layer2(edges_hbm, zw_hbm, zrow_hbm,
               outz_hbm, ridx, cidx, bufa, bufb, zw_s, accz,
               sema, semb, semsa, semsb):
    cid = lax.axis_index("c")
    sid = lax.axis_index("s")
    w = sid * 2 + cid
    e0 = w * _EPW
    r0 = sid * _RPT
    pltpu.sync_copy(zrow_hbm, accz.at[pl.ds(r0, _RPT)])
    pltpu.sync_copy(zw_hbm.at[pl.ds(r0, _RPT)], zw_s.at[pl.ds(r0, _RPT)])
    pltpu.sync_copy(edges_hbm.at[0, pl.ds(e0, _EPW)], ridx)
    pltpu.sync_copy(edges_hbm.at[1, pl.ds(e0, _EPW)], cidx)
    plsc.subcore_barrier()

    def _sem(buf):
        return semsa if buf is bufa else semsb

    def issue(buf, j):
        pltpu.async_copy(buf, accz.at[ridx.at[pl.ds(j * _CHUNK, _CHUNK)]],
                         _sem(buf), add=True)

    def drain(buf, j):
        pltpu.make_async_copy(
            buf, accz.at[ridx.at[pl.ds(j * _CHUNK, _CHUNK)]],
            _sem(buf)).wait()

    _scatter_loop(zw_s, cidx, bufa, bufb, sema, semb, issue, drain)
    plsc.subcore_barrier()
    pltpu.sync_copy(accz.at[pl.ds(r0, _RPT)],
                    outz_hbm.at[pl.ds(r0, _RPT), pl.ds(cid * _Z, _Z)])


def kernel(x, edge_index, W1, gamma1, beta1, mm1, mv1,
           W2, gamma2, beta2, mm2, mv2):
    f32 = jnp.float32
    edges = edge_index.astype(jnp.int32)

    g1 = gamma1.reshape(1, _H).astype(f32)
    b1 = beta1.reshape(1, _H).astype(f32)
    m1 = mm1.reshape(1, _H).astype(f32)
    v1 = mv1.reshape(1, _H).astype(f32)
    g2 = gamma2.reshape(1, _Z).astype(f32)
    b2 = beta2.reshape(1, _Z).astype(f32)
    m2 = mm2.reshape(1, _Z).astype(f32)
    v2 = mv2.reshape(1, _Z).astype(f32)

    zrow_h = jnp.zeros((_RPT, _H), f32)
    zrow_d = jnp.zeros((_RPT, 8), f32)
    zrow_z = jnp.zeros((_RPT, _Z), f32)
    ones8 = jnp.ones((_CHUNK, 8), f32)

    xw = pl.pallas_call(
        _tc1_body,
        out_shape=jax.ShapeDtypeStruct((_N, _H), f32),
    )(x, W1, g1, v1)

    accm, accd = _sc_layer1(edges, xw, zrow_h, zrow_d, ones8)

    zw = pl.pallas_call(
        _tc2_body,
        out_shape=jax.ShapeDtypeStruct((_N, _Z), f32),
    )(accm, accd, W2, g1, b1, m1, v1, g2, v2)

    accz = _sc_layer2(edges, zw, zrow_z)

    z = pl.pallas_call(
        _tc3_body,
        out_shape=jax.ShapeDtypeStruct((_N, _Z), f32),
    )(accz, accd, g2, b2, m2, v2)
    return z


# speedup vs baseline: 1.0349x; 1.0349x over previous
import functools

import jax
import jax.numpy as jnp
from jax import lax
from jax.experimental import pallas as pl
from jax.experimental.pallas import tpu as pltpu
from jax.experimental.pallas import tpu_sc as plsc

_N = 10000
_D = 128
_H = 64
_Z = 32
_E = 320000
_EPS = 1e-3

_NW = 32
_CHUNK = 80
_CH = 125
_EPW = _CH * _CHUNK
_RPT = _N // 16


def _tc1_body(x_ref, w_ref, g_ref, v_ref, o_ref):
    scale = g_ref[...] * lax.rsqrt(v_ref[...] + _EPS)
    o_ref[...] = jnp.dot(x_ref[...], w_ref[...] * scale,
                         preferred_element_type=jnp.float32)


def _tc2_body(am_ref, ad_ref, w2_ref, g1_ref, b1_ref, m1_ref, v1_ref,
              g2_ref, v2_ref, o_ref):
    acc = am_ref[:, 0:_H] + am_ref[:, _H:2 * _H]
    deg = ad_ref[:, 0:1] + ad_ref[:, 8:9]
    inv = jnp.where(deg > 0, 1.0 / deg, 0.0)
    s1 = g1_ref[...] * lax.rsqrt(v1_ref[...] + _EPS)
    sh1 = b1_ref[...] - m1_ref[...] * s1
    h = jnp.maximum(acc * inv + sh1, 0.0)
    s2 = g2_ref[...] * lax.rsqrt(v2_ref[...] + _EPS)
    o_ref[...] = jnp.dot(h, w2_ref[...] * s2,
                         preferred_element_type=jnp.float32)


def _tc3_body(az_ref, ad_ref, g2_ref, b2_ref, m2_ref, v2_ref, o_ref):
    acc = az_ref[:, 0:_Z] + az_ref[:, _Z:2 * _Z]
    deg = ad_ref[:, 0:1] + ad_ref[:, 8:9]
    inv = jnp.where(deg > 0, 1.0 / deg, 0.0)
    s2 = g2_ref[...] * lax.rsqrt(v2_ref[...] + _EPS)
    sh2 = b2_ref[...] - m2_ref[...] * s2
    o_ref[...] = acc * inv + sh2


_mesh = plsc.VectorSubcoreMesh(core_axis_name="c", subcore_axis_name="s")


def _scatter_loop(src_s, ridx, cidx, bufa, bufb, sema, semb, scatter_fn):
    pltpu.async_copy(src_s.at[cidx.at[pl.ds(0, _CHUNK)]], bufa, sema)
    pltpu.async_copy(src_s.at[cidx.at[pl.ds(_CHUNK, _CHUNK)]], bufb, semb)

    def body(i, carry):
        j0 = 2 * i
        pltpu.make_async_copy(
            src_s.at[cidx.at[pl.ds(j0 * _CHUNK, _CHUNK)]], bufa, sema).wait()
        scatter_fn(bufa, j0)

        @pl.when(j0 + 2 < _CH)
        def _():
            pltpu.async_copy(
                src_s.at[cidx.at[pl.ds((j0 + 2) * _CHUNK, _CHUNK)]],
                bufa, sema)

        j1 = j0 + 1
        pltpu.make_async_copy(
            src_s.at[cidx.at[pl.ds(j1 * _CHUNK, _CHUNK)]], bufb, semb).wait()
        scatter_fn(bufb, j1)

        @pl.when(j1 + 2 < _CH)
        def _():
            pltpu.async_copy(
                src_s.at[cidx.at[pl.ds((j1 + 2) * _CHUNK, _CHUNK)]],
                bufb, semb)

        return carry

    lax.fori_loop(0, _CH // 2, body, 0)
    if _CH % 2:
        j = _CH - 1
        pltpu.make_async_copy(
            src_s.at[cidx.at[pl.ds(j * _CHUNK, _CHUNK)]], bufa, sema).wait()
        scatter_fn(bufa, j)


@functools.partial(
    pl.kernel,
    out_type=(
        jax.ShapeDtypeStruct((_N, 2 * _H), jnp.float32),
        jax.ShapeDtypeStruct((_N, 16), jnp.float32),
    ),
    mesh=_mesh,
    scratch_types=[
        pltpu.VMEM((_EPW,), jnp.int32),
        pltpu.VMEM((_EPW,), jnp.int32),
        pltpu.VMEM((_CHUNK, _H), jnp.float32),
        pltpu.VMEM((_CHUNK, _H), jnp.float32),
        pltpu.VMEM((_CHUNK, 8), jnp.float32),
        pltpu.VMEM_SHARED((_N, _H), jnp.float32),
        pltpu.VMEM_SHARED((_N, _H), jnp.float32),
        pltpu.VMEM_SHARED((_N, 8), jnp.float32),
        pltpu.SemaphoreType.DMA,
        pltpu.SemaphoreType.DMA,
    ],
    compiler_params=pltpu.CompilerParams(use_tc_tiling_on_sc=False),
)
def _sc_layer1(edges_hbm, xw_hbm, zrow_hbm, zdeg_hbm, ones_hbm,
               outm_hbm, outd_hbm,
               ridx, cidx, bufa, bufb, ones_v, xw_s, accm, accd, sema, semb):
    cid = lax.axis_index("c")
    sid = lax.axis_index("s")
    w = sid * 2 + cid
    e0 = w * _EPW
    r0 = sid * _RPT
    stage = [
        (zrow_hbm, accm.at[pl.ds(r0, _RPT)], sema),
        (zdeg_hbm, accd.at[pl.ds(r0, _RPT)], semb),
        (xw_hbm.at[pl.ds(r0, _RPT)], xw_s.at[pl.ds(r0, _RPT)], sema),
        (edges_hbm.at[0, pl.ds(e0, _EPW)], ridx, semb),
        (edges_hbm.at[1, pl.ds(e0, _EPW)], cidx, sema),
        (ones_hbm, ones_v, semb),
    ]
    for src, dst, sem in stage:
        pltpu.async_copy(src, dst, sem)
    for src, dst, sem in stage:
        pltpu.make_async_copy(src, dst, sem).wait()
    plsc.subcore_barrier()

    def scatter(buf, j):
        dst = ridx.at[pl.ds(j * _CHUNK, _CHUNK)]
        pltpu.sync_copy(buf, accm.at[dst], add=True)
        pltpu.sync_copy(ones_v, accd.at[dst], add=True)

    _scatter_loop(xw_s, ridx, cidx, bufa, bufb, sema, semb, scatter)
    plsc.subcore_barrier()
    pltpu.sync_copy(accm.at[pl.ds(r0, _RPT)],
                    outm_hbm.at[pl.ds(r0, _RPT), pl.ds(cid * _H, _H)])
    pltpu.sync_copy(accd.at[pl.ds(r0, _RPT)],
                    outd_hbm.at[pl.ds(r0, _RPT), pl.ds(cid * 8, 8)])


@functools.partial(
    pl.kernel,
    out_type=jax.ShapeDtypeStruct((_N, 2 * _Z), jnp.float32),
    mesh=_mesh,
    scratch_types=[
        pltpu.VMEM((_EPW,), jnp.int32),
        pltpu.VMEM((_EPW,), jnp.int32),
        pltpu.VMEM((_CHUNK, _Z), jnp.float32),
        pltpu.VMEM((_CHUNK, _Z), jnp.float32),
        pltpu.VMEM_SHARED((_N, _Z), jnp.float32),
        pltpu.VMEM_SHARED((_N, _Z), jnp.float32),
        pltpu.SemaphoreType.DMA,
        pltpu.SemaphoreType.DMA,
    ],
    compiler_params=pltpu.CompilerParams(use_tc_tiling_on_sc=False),
)
def _sc_layer2(edges_hbm, zw_hbm, zrow_hbm,
               outz_hbm, ridx, cidx, bufa, bufb, zw_s, accz, sema, semb):
    cid = lax.axis_index("c")
    sid = lax.axis_index("s")
    w = sid * 2 + cid
    e0 = w * _EPW
    r0 = sid * _RPT
    stage = [
        (zrow_hbm, accz.at[pl.ds(r0, _RPT)], sema),
        (zw_hbm.at[pl.ds(r0, _RPT)], zw_s.at[pl.ds(r0, _RPT)], semb),
        (edges_hbm.at[0, pl.ds(e0, _EPW)], ridx, sema),
        (edges_hbm.at[1, pl.ds(e0, _EPW)], cidx, semb),
    ]
    for src, dst, sem in stage:
        pltpu.async_copy(src, dst, sem)
    for src, dst, sem in stage:
        pltpu.make_async_copy(src, dst, sem).wait()
    plsc.subcore_barrier()

    def scatter(buf, j):
        dst = ridx.at[pl.ds(j * _CHUNK, _CHUNK)]
        pltpu.sync_copy(buf, accz.at[dst], add=True)

    _scatter_loop(zw_s, ridx, cidx, bufa, bufb, sema, semb, scatter)
    plsc.subcore_barrier()
    pltpu.sync_copy(accz.at[pl.ds(r0, _RPT)],
                    outz_hbm.at[pl.ds(r0, _RPT), pl.ds(cid * _Z, _Z)])


def kernel(x, edge_index, W1, gamma1, beta1, mm1, mv1,
           W2, gamma2, beta2, mm2, mv2):
    f32 = jnp.float32
    edges = edge_index.astype(jnp.int32)

    g1 = gamma1.reshape(1, _H).astype(f32)
    b1 = beta1.reshape(1, _H).astype(f32)
    m1 = mm1.reshape(1, _H).astype(f32)
    v1 = mv1.reshape(1, _H).astype(f32)
    g2 = gamma2.reshape(1, _Z).astype(f32)
    b2 = beta2.reshape(1, _Z).astype(f32)
    m2 = mm2.reshape(1, _Z).astype(f32)
    v2 = mv2.reshape(1, _Z).astype(f32)

    zrow_h = jnp.zeros((_RPT, _H), f32)
    zrow_d = jnp.zeros((_RPT, 8), f32)
    zrow_z = jnp.zeros((_RPT, _Z), f32)
    ones8 = jnp.ones((_CHUNK, 8), f32)

    xw = pl.pallas_call(
        _tc1_body,
        out_shape=jax.ShapeDtypeStruct((_N, _H), f32),
    )(x, W1, g1, v1)

    accm, accd = _sc_layer1(edges, xw, zrow_h, zrow_d, ones8)

    zw = pl.pallas_call(
        _tc2_body,
        out_shape=jax.ShapeDtypeStruct((_N, _Z), f32),
    )(accm, accd, W2, g1, b1, m1, v1, g2, v2)

    accz = _sc_layer2(edges, zw, zrow_z)

    z = pl.pallas_call(
        _tc3_body,
        out_shape=jax.ShapeDtypeStruct((_N, _Z), f32),
    )(accz, accd, g2, b2, m2, v2)
    return z
